# Initial kernel scaffold; baseline (speedup 1.0000x reference)
import jax
import jax.numpy as jnp
from jax.experimental import pallas as pl


def kernel(x, edge_index, W1, b1, W2, b2, W3, b3):
    raise NotImplementedError("write your pallas kernel here")


# speedup vs baseline: 11.2728x; 11.2728x over previous
import functools

import jax
import jax.numpy as jnp
from jax import lax
from jax.experimental import pallas as pl
from jax.experimental.pallas import tpu as pltpu
from jax.experimental.pallas import tpu_sc as plsc

N = 10000
D = 128
E = 320000
NC = 2
NS = 16
NW = NC * NS
EPT = E // NW
CHUNK = 80
NCHUNK = EPT // CHUNK
ZT = 10
ZR = N // ZT
ND_PAD = 10240
DPT = ND_PAD // NS


def _sc_mesh():
    return plsc.VectorSubcoreMesh(core_axis_name="c", subcore_axis_name="s")


def _sc_degree(dst, zdeg):
    @functools.partial(
        pl.kernel,
        out_type=jax.ShapeDtypeStruct((NC * ND_PAD,), jnp.float32),
        mesh=_sc_mesh(),
        scratch_types=[
            pltpu.VMEM((CHUNK,), jnp.int32),
            pltpu.VMEM((CHUNK,), jnp.float32),
            pltpu.VMEM_SHARED((ND_PAD,), jnp.float32),
        ],
    )
    def run(dst_hbm, z_hbm, out_hbm, didx, ones, acc):
        c = lax.axis_index("c")
        s = lax.axis_index("s")
        wid = c * NS + s
        pltpu.sync_copy(z_hbm, acc.at[pl.ds(s * DPT, DPT)])
        for j in range(CHUNK // 16):
            ones[pl.ds(j * 16, 16)] = jnp.full((16,), 1.0, jnp.float32)
        plsc.subcore_barrier()

        @pl.loop(0, NCHUNK)
        def _(i):
            base = wid * EPT + i * CHUNK
            pltpu.sync_copy(dst_hbm.at[pl.ds(base, CHUNK)], didx)
            pltpu.sync_copy(ones, acc.at[didx], add=True)

        plsc.subcore_barrier()
        pltpu.sync_copy(acc.at[pl.ds(s * DPT, DPT)],
                        out_hbm.at[pl.ds(c * ND_PAD + s * DPT, DPT)])

    return run(dst, zdeg)


def _sc_aggregate(g, src, dst, zrows):
    @functools.partial(
        pl.kernel,
        out_type=jax.ShapeDtypeStruct((NC * N, D), jnp.float32),
        mesh=_sc_mesh(),
        scratch_types=[
            pltpu.VMEM((CHUNK,), jnp.int32),
            pltpu.VMEM((CHUNK,), jnp.int32),
            pltpu.VMEM((CHUNK, D), jnp.float32),
            pltpu.VMEM_SHARED((N, D), jnp.float32),
            pltpu.SemaphoreType.DMA,
        ],
    )
    def run(g_hbm, src_hbm, dst_hbm, z_hbm, out_hbm, sidx, didx, rows, acc, sem):
        c = lax.axis_index("c")
        s = lax.axis_index("s")
        wid = c * NS + s

        @pl.when(s < ZT)
        def _():
            pltpu.sync_copy(z_hbm, acc.at[pl.ds(s * ZR, ZR)])

        plsc.subcore_barrier()

        @pl.loop(0, NCHUNK)
        def _(i):
            base = wid * EPT + i * CHUNK
            pltpu.sync_copy(src_hbm.at[pl.ds(base, CHUNK)], sidx)
            pltpu.sync_copy(dst_hbm.at[pl.ds(base, CHUNK)], didx)
            pltpu.async_copy(g_hbm.at[sidx], rows, sem).wait()
            pltpu.sync_copy(rows, acc.at[didx], add=True)

        plsc.subcore_barrier()

        @pl.when(s < ZT)
        def _():
            pltpu.sync_copy(acc.at[pl.ds(s * ZR, ZR)],
                            out_hbm.at[pl.ds(c * N + s * ZR, ZR)])

    return run(g, src, dst, zrows)


R = 1000
G = N // R


def _row_spec():
    return pl.BlockSpec((R, D), lambda i: (i, 0))


def _deg_spec():
    return pl.BlockSpec((R, 2), lambda i: (i, 0))


def _w_spec():
    return pl.BlockSpec((D, D), lambda i: (0, 0))


def _b_spec():
    return pl.BlockSpec((1, D), lambda i: (0, 0))


def _dinv_of(deg_blk):
    return lax.rsqrt(deg_blk[:, 0] + deg_blk[:, 1] + 1.0)


def _tc_pre(deg2, x, w):
    def body(deg_ref, x_ref, w_ref, g_ref):
        dinv = _dinv_of(deg_ref[...])
        h = jnp.dot(x_ref[...], w_ref[...], preferred_element_type=jnp.float32)
        g_ref[...] = h * dinv[:, None]

    return pl.pallas_call(
        body,
        grid=(G,),
        in_specs=[_deg_spec(), _row_spec(), _w_spec()],
        out_specs=_row_spec(),
        out_shape=jax.ShapeDtypeStruct((N, D), jnp.float32),
    )(deg2, x, w)


def _tc_mid(deg2, parts, gprev, b, w):
    def body(deg_ref, p0_ref, p1_ref, gp_ref, b_ref, w_ref, g_ref):
        dinv = _dinv_of(deg_ref[...])
        t = (p0_ref[...] + p1_ref[...] + gp_ref[...]) * dinv[:, None] + b_ref[...]
        z = jnp.maximum(t, 0.0)
        h = jnp.dot(z, w_ref[...], preferred_element_type=jnp.float32)
        g_ref[...] = h * dinv[:, None]

    p1_spec = pl.BlockSpec((R, D), lambda i: (G + i, 0))
    return pl.pallas_call(
        body,
        grid=(G,),
        in_specs=[_deg_spec(), _row_spec(), p1_spec, _row_spec(), _b_spec(), _w_spec()],
        out_specs=_row_spec(),
        out_shape=jax.ShapeDtypeStruct((N, D), jnp.float32),
    )(deg2, parts, parts, gprev, b, w)


def _tc_post(deg2, parts, g3, b):
    def body(deg_ref, p0_ref, p1_ref, g_ref, b_ref, o_ref):
        dinv = _dinv_of(deg_ref[...])
        o_ref[...] = (p0_ref[...] + p1_ref[...] + g_ref[...]) * dinv[:, None] + b_ref[...]

    p1_spec = pl.BlockSpec((R, D), lambda i: (G + i, 0))
    return pl.pallas_call(
        body,
        grid=(G,),
        in_specs=[_deg_spec(), _row_spec(), p1_spec, _row_spec(), _b_spec()],
        out_specs=_row_spec(),
        out_shape=jax.ShapeDtypeStruct((N, D), jnp.float32),
    )(deg2, parts, parts, g3, b)


def kernel(x, edge_index, W1, b1, W2, b2, W3, b3):
    ei = edge_index.astype(jnp.int32)
    src, dst = ei[0], ei[1]
    zdeg = jnp.zeros((DPT,), jnp.float32)
    zrows = jnp.zeros((ZR, D), jnp.float32)

    degp = _sc_degree(dst, zdeg)
    deg2 = degp.reshape(NC, ND_PAD)[:, :N].T

    b1r = b1.reshape(1, D)
    b2r = b2.reshape(1, D)
    b3r = b3.reshape(1, D)

    g1 = _tc_pre(deg2, x, W1)
    p = _sc_aggregate(g1, src, dst, zrows)
    g2 = _tc_mid(deg2, p, g1, b1r, W2)
    q = _sc_aggregate(g2, src, dst, zrows)
    g3 = _tc_mid(deg2, q, g2, b2r, W3)
    r = _sc_aggregate(g3, src, dst, zrows)
    return _tc_post(deg2, r, g3, b3r)
